# baseline (speedup 1.0000x reference)
import jax
import jax.numpy as jnp
from jax import lax
from jax.experimental import pallas as pl
from jax.experimental.pallas import tpu as pltpu
from jax.experimental.pallas import tpu_sc as plsc

L = 16
NCHUNK = 7
NPAD = NCHUNK * L
NBOX = 100
NIMG = 8
IOU_THR = 0.5
IN_W = 5 * NPAD
OUT_W = 5 * NPAD + L


def _nms_body(in_hbm, out_hbm, io_v, o_v, area_v, keep_v):
    wid = lax.axis_index("s") * 2 + lax.axis_index("c")

    @pl.when(wid < NIMG)
    def _():
        pltpu.sync_copy(in_hbm.at[wid], io_v)

        iota = lax.iota(jnp.int32, L)
        gidx = [iota + iv * L for iv in range(NCHUNK)]
        zeros = jnp.zeros((L,), jnp.int32)
        zf = jnp.zeros((L,), jnp.float32)
        svecs = [io_v[pl.ds(iv * L, L)] for iv in range(NCHUNK)]

        def rank_chunk(jc, ranks):
            sjv = io_v[pl.ds(jc * L, L)]
            for l in range(L):
                j = jc * L + l
                sj = zf + sjv[l]
                ranks = tuple(
                    ranks[iv]
                    + ((sj > svecs[iv])
                       | ((sj == svecs[iv]) & (j < gidx[iv]))).astype(jnp.int32)
                    for iv in range(NCHUNK))
            return ranks

        ranks = lax.fori_loop(0, NCHUNK - 1, rank_chunk,
                              tuple(zeros for _ in range(NCHUNK)))
        sjv6 = svecs[NCHUNK - 1]
        for l in range(NBOX - (NCHUNK - 1) * L):
            j = (NCHUNK - 1) * L + l
            sj = zf + sjv6[l]
            ranks = tuple(
                ranks[iv]
                + ((sj > svecs[iv])
                   | ((sj == svecs[iv]) & (j < gidx[iv]))).astype(jnp.int32)
                for iv in range(NCHUNK))
        last = NCHUNK - 1
        ranks = ranks[:last] + (
            jnp.where(gidx[last] >= NBOX, gidx[last], ranks[last]),)

        for iv in range(NCHUNK):
            r = ranks[iv]
            for k in range(5):
                plsc.store_scatter(o_v, [r + k * NPAD],
                                   io_v[pl.ds(k * NPAD + iv * L, L)])

        ones = jnp.ones((L,), jnp.int32)
        for iv in range(NCHUNK):
            sl = pl.ds(iv * L, L)
            w = jnp.maximum(o_v[pl.ds(3 * NPAD + iv * L, L)]
                            - o_v[pl.ds(1 * NPAD + iv * L, L)], 0.0)
            h = jnp.maximum(o_v[pl.ds(4 * NPAD + iv * L, L)]
                            - o_v[pl.ds(2 * NPAD + iv * L, L)], 0.0)
            area_v[sl] = w * h
            keep_v[sl] = ones

        def nms_chunk(ic, carry):
            base = ic * L
            cx1 = o_v[pl.ds(1 * NPAD + base, L)]
            cy1 = o_v[pl.ds(2 * NPAD + base, L)]
            cx2 = o_v[pl.ds(3 * NPAD + base, L)]
            cy2 = o_v[pl.ds(4 * NPAD + base, L)]
            car = area_v[pl.ds(base, L)]
            for l in range(L):
                i = base + l
                kc = keep_v[pl.ds(base, L)]
                @pl.when((kc[l] != 0) & (i < NBOX))
                def _(l=l, i=i, cx1=cx1, cy1=cy1, cx2=cx2, cy2=cy2, car=car):
                    xi1 = zf + cx1[l]
                    yi1 = zf + cy1[l]
                    xi2 = zf + cx2[l]
                    yi2 = zf + cy2[l]
                    ai = zf + car[l]
                    for jv in range(NCHUNK):
                        @pl.when(jv * L + (L - 1) > i)
                        def _(jv=jv):
                            sl = pl.ds(jv * L, L)
                            xx1 = jnp.maximum(o_v[pl.ds(1 * NPAD + jv * L, L)], xi1)
                            yy1 = jnp.maximum(o_v[pl.ds(2 * NPAD + jv * L, L)], yi1)
                            xx2 = jnp.minimum(o_v[pl.ds(3 * NPAD + jv * L, L)], xi2)
                            yy2 = jnp.minimum(o_v[pl.ds(4 * NPAD + jv * L, L)], yi2)
                            inter = (jnp.maximum(xx2 - xx1, 0.0) *
                                     jnp.maximum(yy2 - yy1, 0.0))
                            union = ai + area_v[sl] - inter
                            iou = inter / jnp.maximum(union, 1e-9)
                            supp = (iou > IOU_THR) & (gidx[jv] > i)
                            keep_v[sl] = jnp.where(supp, 0, keep_v[sl])
            return carry

        lax.fori_loop(0, NCHUNK, nms_chunk, 0)

        total = jnp.int32(0)
        for iv in range(NCHUNK):
            sl = pl.ds(iv * L, L)
            kv = keep_v[sl]
            total = total + jnp.sum(kv * (gidx[iv] < NBOX).astype(jnp.int32))
            kf = kv.astype(jnp.float32)
            for k in range(5):
                slk = pl.ds(k * NPAD + iv * L, L)
                o_v[slk] = o_v[slk] * kf
        o_v[pl.ds(5 * NPAD, L)] = zf + total.astype(jnp.float32)

        pltpu.sync_copy(o_v, out_hbm.at[wid])


def kernel(x, boxes, scores, W1, b1, W2, b2, Wb, bb, Wc, bc):
    del x, W1, b1, W2, b2, Wb, bb, Wc, bc
    nb, nn = scores.shape
    pad = NPAD - nn
    sc_p = jnp.pad(scores, ((0, 0), (0, pad)), constant_values=-1.0)
    bx_p = jnp.pad(boxes, ((0, 0), (0, pad), (0, 0)))
    packed = jnp.concatenate(
        [sc_p] + [bx_p[:, :, k] for k in range(4)], axis=1)

    mesh = plsc.VectorSubcoreMesh(core_axis_name="c", subcore_axis_name="s",
                                  num_cores=2, num_subcores=16)
    f32 = jnp.float32
    out = pl.kernel(
        _nms_body,
        out_type=jax.ShapeDtypeStruct((nb, OUT_W), f32),
        mesh=mesh,
        compiler_params=pltpu.CompilerParams(needs_layout_passes=False),
        scratch_types=[
            pltpu.VMEM((IN_W,), f32),
            pltpu.VMEM((OUT_W,), f32),
            pltpu.VMEM((NPAD,), f32),
            pltpu.VMEM((NPAD,), jnp.int32),
        ],
    )(packed)

    final_scores = out[:, :nn]
    final_boxes = out[:, NPAD:5 * NPAD].reshape(nb, 4, NPAD)
    final_boxes = final_boxes.transpose(0, 2, 1)[:, :nn]
    num_detections = out[:, 5 * NPAD].astype(jnp.int32)
    return final_boxes, final_scores, num_detections


# speedup vs baseline: 2.2431x; 2.2431x over previous
import jax
import jax.numpy as jnp
from jax import lax
from jax.experimental import pallas as pl
from jax.experimental.pallas import tpu as pltpu
from jax.experimental.pallas import tpu_sc as plsc

L = 16
NCHUNK = 7
NPAD = NCHUNK * L
NBOX = 100
NIMG = 8
IOU_THR = 0.5
IN_W = 5 * NPAD
OUT_W = 5 * NPAD + L
ACW = NPAD + L


def _nms_body(in_hbm, out_hbm, io_v, o_v, area_v, keep_v,
              acx1_v, acy1_v, acx2_v, acy2_v, acar_v, acidx_v, keepc_v):
    wid = lax.axis_index("s") * 2 + lax.axis_index("c")

    @pl.when(wid < NIMG)
    def _():
        pltpu.sync_copy(in_hbm.at[wid], io_v)

        iota = lax.iota(jnp.int32, L)
        gidx = [iota + iv * L for iv in range(NCHUNK)]
        zeros = jnp.zeros((L,), jnp.int32)
        svecs = [io_v[pl.ds(iv * L, L)] for iv in range(NCHUNK)]

        def rank_body(j, ranks):
            sj = plsc.load_gather(io_v, [zeros + j])
            out = []
            for iv in range(NCHUNK):
                beats = (sj > svecs[iv]) | ((sj == svecs[iv]) & (j < gidx[iv]))
                out.append(ranks[iv] + beats.astype(jnp.int32))
            return tuple(out)

        ranks = lax.fori_loop(0, NBOX, rank_body,
                              tuple(zeros for _ in range(NCHUNK)))
        last = NCHUNK - 1
        ranks = ranks[:last] + (
            jnp.where(gidx[last] >= NBOX, gidx[last], ranks[last]),)

        for iv in range(NCHUNK):
            r = ranks[iv]
            for k in range(5):
                plsc.store_scatter(o_v, [r + k * NPAD],
                                   io_v[pl.ds(k * NPAD + iv * L, L)])

        ones = jnp.ones((L,), jnp.int32)
        n_act = jnp.int32(0)
        for iv in range(NCHUNK):
            sl = pl.ds(iv * L, L)
            x1c = o_v[pl.ds(1 * NPAD + iv * L, L)]
            y1c = o_v[pl.ds(2 * NPAD + iv * L, L)]
            x2c = o_v[pl.ds(3 * NPAD + iv * L, L)]
            y2c = o_v[pl.ds(4 * NPAD + iv * L, L)]
            ar = (jnp.maximum(x2c - x1c, 0.0) *
                  jnp.maximum(y2c - y1c, 0.0))
            area_v[sl] = ar
            keep_v[sl] = ones
            act = (x2c > x1c) & (y2c > y1c)
            dst = pl.ds(n_act, L)
            plsc.store_compressed(acx1_v.at[dst], x1c, mask=act)
            plsc.store_compressed(acy1_v.at[dst], y1c, mask=act)
            plsc.store_compressed(acx2_v.at[dst], x2c, mask=act)
            plsc.store_compressed(acy2_v.at[dst], y2c, mask=act)
            plsc.store_compressed(acar_v.at[dst], ar, mask=act)
            plsc.store_compressed(acidx_v.at[dst], gidx[iv], mask=act)
            n_act = n_act + jnp.sum(act.astype(jnp.int32))
        for iv in range(NCHUNK + 1):
            keepc_v[pl.ds(iv * L, L)] = ones

        def nms_body(t, carry):
            ts = zeros + t
            alive = plsc.load_gather(keepc_v, [ts]) != 0
            xi1 = plsc.load_gather(acx1_v, [ts])
            yi1 = plsc.load_gather(acy1_v, [ts])
            xi2 = plsc.load_gather(acx2_v, [ts])
            yi2 = plsc.load_gather(acy2_v, [ts])
            ai = plsc.load_gather(acar_v, [ts])
            for jc in range(NCHUNK):
                @pl.when((jc * L < n_act) & (jc * L + (L - 1) > t))
                def _(jc=jc):
                    sl = pl.ds(jc * L, L)
                    xx1 = jnp.maximum(acx1_v[sl], xi1)
                    yy1 = jnp.maximum(acy1_v[sl], yi1)
                    xx2 = jnp.minimum(acx2_v[sl], xi2)
                    yy2 = jnp.minimum(acy2_v[sl], yi2)
                    inter = (jnp.maximum(xx2 - xx1, 0.0) *
                             jnp.maximum(yy2 - yy1, 0.0))
                    union = ai + acar_v[sl] - inter
                    iou = inter / jnp.maximum(union, 1e-9)
                    supp = (iou > IOU_THR) & (gidx[jc] > t) & alive
                    keepc_v[sl] = jnp.where(supp, 0, keepc_v[sl])
            return carry

        lax.fori_loop(0, n_act, nms_body, 0)

        for jc in range(NCHUNK):
            @pl.when(jc * L < n_act)
            def _(jc=jc):
                sl = pl.ds(jc * L, L)
                plsc.store_scatter(keep_v, [acidx_v[sl]], keepc_v[sl],
                                   mask=gidx[jc] < n_act)

        total = jnp.int32(0)
        for iv in range(NCHUNK):
            sl = pl.ds(iv * L, L)
            kv = keep_v[sl]
            total = total + jnp.sum(kv * (gidx[iv] < NBOX).astype(jnp.int32))
            kf = kv.astype(jnp.float32)
            for k in range(5):
                slk = pl.ds(k * NPAD + iv * L, L)
                o_v[slk] = o_v[slk] * kf
        zf = jnp.zeros((L,), jnp.float32)
        o_v[pl.ds(5 * NPAD, L)] = zf + total.astype(jnp.float32)

        pltpu.sync_copy(o_v, out_hbm.at[wid])


def kernel(x, boxes, scores, W1, b1, W2, b2, Wb, bb, Wc, bc):
    del x, W1, b1, W2, b2, Wb, bb, Wc, bc
    nb, nn = scores.shape
    pad = NPAD - nn
    sc_p = jnp.pad(scores, ((0, 0), (0, pad)), constant_values=-1.0)
    bx_p = jnp.pad(boxes, ((0, 0), (0, pad), (0, 0)))
    packed = jnp.concatenate(
        [sc_p] + [bx_p[:, :, k] for k in range(4)], axis=1)

    mesh = plsc.VectorSubcoreMesh(core_axis_name="c", subcore_axis_name="s",
                                  num_cores=2, num_subcores=16)
    f32 = jnp.float32
    out = pl.kernel(
        _nms_body,
        out_type=jax.ShapeDtypeStruct((nb, OUT_W), f32),
        mesh=mesh,
        compiler_params=pltpu.CompilerParams(needs_layout_passes=False),
        scratch_types=[
            pltpu.VMEM((IN_W,), f32),
            pltpu.VMEM((OUT_W,), f32),
            pltpu.VMEM((NPAD,), f32),
            pltpu.VMEM((NPAD,), jnp.int32),
            pltpu.VMEM((ACW,), f32),
            pltpu.VMEM((ACW,), f32),
            pltpu.VMEM((ACW,), f32),
            pltpu.VMEM((ACW,), f32),
            pltpu.VMEM((ACW,), f32),
            pltpu.VMEM((ACW,), jnp.int32),
            pltpu.VMEM((ACW,), jnp.int32),
        ],
    )(packed)

    final_scores = out[:, :nn]
    final_boxes = out[:, NPAD:5 * NPAD].reshape(nb, 4, NPAD)
    final_boxes = final_boxes.transpose(0, 2, 1)[:, :nn]
    num_detections = out[:, 5 * NPAD].astype(jnp.int32)
    return final_boxes, final_scores, num_detections
